# baseline (speedup 1.0000x reference)
import functools

import jax
import jax.numpy as jnp
from jax import lax
from jax.experimental import pallas as pl
from jax.experimental.pallas import tpu as pltpu
from jax.experimental.pallas import tpu_sc as plsc

B = 4096
K = 64
DIM = 128
N_R_PAD = 128
N_T_PAD = 9472

NC = 2
NS = 16
NW = NC * NS
BPW = B // NW
CHUNK = 2
NCHUNKS = BPW // CHUNK
ROWS = CHUNK * K
NBUF = 4


def _score_matmul(u_emb, r_pad):
    def mm(u_ref, r_ref, o_ref):
        o_ref[...] = lax.dot_general(
            u_ref[...], r_ref[...], (((1,), (1,)), ((), ())),
            preferred_element_type=jnp.float32,
            precision=lax.Precision.HIGHEST)

    return pl.pallas_call(
        mm,
        out_shape=jax.ShapeDtypeStruct((u_emb.shape[0], N_R_PAD), jnp.float32),
    )(u_emb, r_pad)


def _make_sc_kernel():
    mesh = plsc.VectorSubcoreMesh(core_axis_name="c", subcore_axis_name="s")

    @functools.partial(
        pl.kernel, mesh=mesh,
        compiler_params=pltpu.CompilerParams(
            needs_layout_passes=False, use_tc_tiling_on_sc=False),
        out_type=jax.ShapeDtypeStruct((B * DIM,), jnp.float32),
        scratch_types=[
            pltpu.VMEM((BPW,), jnp.int32),
            pltpu.VMEM((BPW * K,), jnp.int32),
            pltpu.VMEM((NCHUNKS, ROWS), jnp.int32),
            pltpu.VMEM((BPW, N_R_PAD), jnp.float32),
            pltpu.VMEM((ROWS, DIM // 2), jnp.int32),
            pltpu.VMEM((ROWS, DIM // 2), jnp.int32),
            pltpu.VMEM((ROWS, DIM // 2), jnp.int32),
            pltpu.VMEM((ROWS, DIM // 2), jnp.int32),
            pltpu.VMEM((K,), jnp.float32),
            pltpu.VMEM((BPW * DIM,), jnp.float32),
            pltpu.VMEM_SHARED((N_T_PAD, DIM // 2), jnp.int32),
            pltpu.SemaphoreType.DMA,
            pltpu.SemaphoreType.DMA,
            pltpu.SemaphoreType.DMA,
            pltpu.SemaphoreType.DMA,
        ],
    )
    def sc_kernel(s_hbm, u_hbm, kgr_hbm, kgt_hbm, t_hbm, out_hbm,
                  u_v, kgr_v, kgt_v, s_rows, t_rows0, t_rows1, t_rows2,
                  t_rows3, w_v, out_v, t_spm, sem0, sem1, sem2, sem3):
        wid = lax.axis_index("s") * NC + lax.axis_index("c")
        sid = lax.axis_index("s")
        bufs = (t_rows0, t_rows1, t_rows2, t_rows3)
        sems = (sem0, sem1, sem2, sem3)
        tpt = N_T_PAD // NS
        pltpu.sync_copy(t_hbm.at[pl.ds(sid * tpt, tpt)],
                        t_spm.at[pl.ds(sid * tpt, tpt)])
        pltpu.sync_copy(u_hbm.at[pl.ds(wid * BPW, BPW)], u_v)
        pltpu.sync_copy(kgr_hbm.at[pl.ds(wid * BPW * K, BPW * K)], kgr_v)
        pltpu.sync_copy(kgt_hbm.at[pl.ds(wid * NCHUNKS, NCHUNKS)], kgt_v)
        pltpu.async_copy(s_hbm.at[u_v], s_rows, sem0).wait()
        plsc.subcore_barrier()

        def start_gather(c, buf, sem):
            pltpu.make_async_copy(t_spm.at[kgt_v.at[c]], buf, sem).start()

        def wait_gather(c, buf, sem):
            pltpu.make_async_copy(t_spm.at[kgt_v.at[c]], buf, sem).wait()

        for p in range(NBUF - 1):
            start_gather(p, bufs[p], sems[p])

        def chunk_quad_body(c4, carry):
            for b in range(NBUF):
                c = c4 * NBUF + b
                nxt = jnp.minimum(c + NBUF - 1, NCHUNKS - 1)
                pb = (b + NBUF - 1) % NBUF
                start_gather(nxt, bufs[pb], sems[pb])
                wait_gather(c, bufs[b], sems[b])
                _compute_chunk(c, bufs[b])
            return carry

        def _compute_chunk(c, t_rows):
            for ci in range(CHUNK):
                i = c * CHUNK + ci
                svecs = []
                for c16 in range(4):
                    kvec = kgr_v[pl.ds(i * K + c16 * 16, 16)]
                    iv = jnp.full((16,), i, jnp.int32)
                    svecs.append(plsc.load_gather(s_rows, [iv, kvec]))
                m = jnp.maximum(jnp.maximum(svecs[0], svecs[1]),
                                jnp.maximum(svecs[2], svecs[3]))
                mx = jnp.max(m)
                evecs = [jnp.exp(sv - mx) for sv in svecs]
                tot = (jnp.sum(evecs[0]) + jnp.sum(evecs[1])
                       + jnp.sum(evecs[2]) + jnp.sum(evecs[3]))
                inv = jnp.ones((16,), jnp.float32) / jnp.full((16,), tot)
                for c16 in range(4):
                    w_v[pl.ds(c16 * 16, 16)] = evecs[c16] * inv

                def row_body(kk, acc):
                    wkv = plsc.load_gather(
                        w_v, [jnp.full((16,), kk, jnp.int32)])
                    wkb = plsc.pack(wkv, wkv,
                                    format=plsc.PackFormat.INTERLEAVED)
                    r = ci * K + kk
                    new = []
                    for q in range(4):
                        tq = plsc.bitcast(
                            t_rows[r, pl.ds(q * 16, 16)], jnp.bfloat16)
                        ev, od = plsc.unpack(
                            wkb * tq, format=plsc.PackFormat.INTERLEAVED,
                            preferred_element_type=jnp.float32)
                        new.append(acc[2 * q] + ev)
                        new.append(acc[2 * q + 1] + od)
                    return tuple(new)

                zeros8 = tuple(jnp.zeros((16,), jnp.float32) for _ in range(8))
                acc = lax.fori_loop(0, K, row_body, zeros8, unroll=2)
                lanes2 = 2 * lax.iota(jnp.int32, 16)
                for q in range(4):
                    base = i * DIM + q * 32
                    plsc.store_scatter(out_v, [base + lanes2], acc[2 * q])
                    plsc.store_scatter(out_v, [base + 1 + lanes2],
                                       acc[2 * q + 1])

        lax.fori_loop(0, NCHUNKS // NBUF, chunk_quad_body, 0)
        for p in range(NBUF - 1):
            wait_gather(NCHUNKS - 1, bufs[p], sems[p])
        pltpu.sync_copy(out_v, out_hbm.at[pl.ds(wid * BPW * DIM, BPW * DIM)])

    return sc_kernel


_sc_kernel = _make_sc_kernel()


def kernel(u, v, kg_R, kg_T, T_emb, R_emb, U_emb):
    r_pad = jnp.zeros((N_R_PAD, DIM), jnp.float32).at[:R_emb.shape[0]].set(R_emb)
    s_full = _score_matmul(U_emb, r_pad)
    kgr_flat = kg_R.reshape(-1)
    kgt_resh = kg_T.reshape(NW * NCHUNKS, ROWS)
    t_bf = jnp.zeros((N_T_PAD, DIM), jnp.bfloat16).at[:T_emb.shape[0]].set(
        T_emb.astype(jnp.bfloat16))
    t_i32 = lax.bitcast_convert_type(
        t_bf.reshape(N_T_PAD, DIM // 2, 2), jnp.int32)
    out_flat = _sc_kernel(s_full, u, kgr_flat, kgt_resh, t_i32)
    return out_flat.reshape(B, DIM)


# speedup vs baseline: 1.0054x; 1.0054x over previous
import functools

import jax
import jax.numpy as jnp
from jax import lax
from jax.experimental import pallas as pl
from jax.experimental.pallas import tpu as pltpu
from jax.experimental.pallas import tpu_sc as plsc

B = 4096
K = 64
DIM = 128
N_R_PAD = 128
N_T_PAD = 9472

NC = 2
NS = 16
NW = NC * NS
BPW = B // NW
CHUNK = 2
NCHUNKS = BPW // CHUNK
ROWS = CHUNK * K
NBUF = 2


def _score_matmul(u_emb, r_pad):
    def mm(u_ref, r_ref, o_ref):
        o_ref[...] = lax.dot_general(
            u_ref[...], r_ref[...], (((1,), (1,)), ((), ())),
            preferred_element_type=jnp.float32,
            precision=lax.Precision.HIGHEST)

    return pl.pallas_call(
        mm,
        out_shape=jax.ShapeDtypeStruct((u_emb.shape[0], N_R_PAD), jnp.float32),
    )(u_emb, r_pad)


def _make_sc_kernel():
    mesh = plsc.VectorSubcoreMesh(core_axis_name="c", subcore_axis_name="s")

    @functools.partial(
        pl.kernel, mesh=mesh,
        compiler_params=pltpu.CompilerParams(
            needs_layout_passes=False, use_tc_tiling_on_sc=False),
        out_type=jax.ShapeDtypeStruct((B * DIM,), jnp.float32),
        scratch_types=[
            pltpu.VMEM((BPW,), jnp.int32),
            pltpu.VMEM((BPW * K,), jnp.int32),
            pltpu.VMEM((NCHUNKS, ROWS), jnp.int32),
            pltpu.VMEM((BPW, N_R_PAD), jnp.float32),
            pltpu.VMEM((ROWS, DIM // 2), jnp.int32),
            pltpu.VMEM((ROWS, DIM // 2), jnp.int32),
            pltpu.VMEM((K,), jnp.float32),
            pltpu.VMEM((BPW * DIM,), jnp.float32),
            pltpu.VMEM_SHARED((N_T_PAD, DIM // 2), jnp.int32),
            pltpu.SemaphoreType.DMA,
            pltpu.SemaphoreType.DMA,
        ],
    )
    def sc_kernel(s_hbm, u_hbm, kgr_hbm, kgt_hbm, t_hbm, out_hbm,
                  u_v, kgr_v, kgt_v, s_rows, t_rows0, t_rows1,
                  w_v, out_v, t_spm, sem0, sem1):
        wid = lax.axis_index("s") * NC + lax.axis_index("c")
        sid = lax.axis_index("s")
        bufs = (t_rows0, t_rows1)
        sems = (sem0, sem1)
        tpt = N_T_PAD // NS
        pltpu.sync_copy(t_hbm.at[pl.ds(sid * tpt, tpt)],
                        t_spm.at[pl.ds(sid * tpt, tpt)])
        pltpu.sync_copy(u_hbm.at[pl.ds(wid * BPW, BPW)], u_v)
        pltpu.sync_copy(kgr_hbm.at[pl.ds(wid * BPW * K, BPW * K)], kgr_v)
        pltpu.sync_copy(kgt_hbm.at[pl.ds(wid * NCHUNKS, NCHUNKS)], kgt_v)
        pltpu.async_copy(s_hbm.at[u_v], s_rows, sem0).wait()
        plsc.subcore_barrier()

        def start_gather(c, buf, sem):
            pltpu.make_async_copy(t_spm.at[kgt_v.at[c]], buf, sem).start()

        def wait_gather(c, buf, sem):
            pltpu.make_async_copy(t_spm.at[kgt_v.at[c]], buf, sem).wait()

        for p in range(NBUF - 1):
            start_gather(p, bufs[p], sems[p])

        def chunk_quad_body(c4, carry):
            for b in range(NBUF):
                c = c4 * NBUF + b
                nxt = jnp.minimum(c + NBUF - 1, NCHUNKS - 1)
                pb = (b + NBUF - 1) % NBUF
                start_gather(nxt, bufs[pb], sems[pb])
                wait_gather(c, bufs[b], sems[b])
                _compute_chunk(c, bufs[b])
            return carry

        def _compute_chunk(c, t_rows):
            for ci in range(CHUNK):
                i = c * CHUNK + ci
                svecs = []
                for c16 in range(4):
                    kvec = kgr_v[pl.ds(i * K + c16 * 16, 16)]
                    iv = jnp.full((16,), i, jnp.int32)
                    svecs.append(plsc.load_gather(s_rows, [iv, kvec]))
                m = jnp.maximum(jnp.maximum(svecs[0], svecs[1]),
                                jnp.maximum(svecs[2], svecs[3]))
                mx = jnp.max(m)
                evecs = [jnp.exp(sv - mx) for sv in svecs]
                tot = (jnp.sum(evecs[0]) + jnp.sum(evecs[1])
                       + jnp.sum(evecs[2]) + jnp.sum(evecs[3]))
                inv = jnp.ones((16,), jnp.float32) / jnp.full((16,), tot)
                for c16 in range(4):
                    w_v[pl.ds(c16 * 16, 16)] = evecs[c16] * inv

                def row_body(kk, acc):
                    wkv = plsc.load_gather(
                        w_v, [jnp.full((16,), kk, jnp.int32)])
                    wkb = plsc.pack(wkv, wkv,
                                    format=plsc.PackFormat.INTERLEAVED)
                    r = ci * K + kk
                    new = []
                    for q in range(4):
                        tq = plsc.bitcast(
                            t_rows[r, pl.ds(q * 16, 16)], jnp.bfloat16)
                        ev, od = plsc.unpack(
                            wkb * tq, format=plsc.PackFormat.INTERLEAVED,
                            preferred_element_type=jnp.float32)
                        new.append(acc[2 * q] + ev)
                        new.append(acc[2 * q + 1] + od)
                    return tuple(new)

                zeros8 = tuple(jnp.zeros((16,), jnp.float32) for _ in range(8))
                acc = lax.fori_loop(0, K, row_body, zeros8, unroll=2)
                lanes2 = 2 * lax.iota(jnp.int32, 16)
                for q in range(4):
                    base = i * DIM + q * 32
                    plsc.store_scatter(out_v, [base + lanes2], acc[2 * q])
                    plsc.store_scatter(out_v, [base + 1 + lanes2],
                                       acc[2 * q + 1])

        lax.fori_loop(0, NCHUNKS // NBUF, chunk_quad_body, 0)
        for p in range(NBUF - 1):
            wait_gather(NCHUNKS - 1, bufs[p], sems[p])
        pltpu.sync_copy(out_v, out_hbm.at[pl.ds(wid * BPW * DIM, BPW * DIM)])

    return sc_kernel


_sc_kernel = _make_sc_kernel()


def kernel(u, v, kg_R, kg_T, T_emb, R_emb, U_emb):
    r_pad = jnp.zeros((N_R_PAD, DIM), jnp.float32).at[:R_emb.shape[0]].set(R_emb)
    s_full = _score_matmul(U_emb, r_pad)
    kgr_flat = kg_R.reshape(-1)
    kgt_resh = kg_T.reshape(NW * NCHUNKS, ROWS)
    t_bf = jnp.zeros((N_T_PAD, DIM), jnp.bfloat16).at[:T_emb.shape[0]].set(
        T_emb.astype(jnp.bfloat16))
    t_i32 = lax.bitcast_convert_type(
        t_bf.reshape(N_T_PAD, DIM // 2, 2), jnp.int32)
    out_flat = _sc_kernel(s_full, u, kgr_flat, kgt_resh, t_i32)
    return out_flat.reshape(B, DIM)
